# Initial kernel scaffold; baseline (speedup 1.0000x reference)
import jax
import jax.numpy as jnp
from jax.experimental import pallas as pl


def kernel(pred, gt):
    raise NotImplementedError("write your pallas kernel here")


# speedup vs baseline: 255.3932x; 255.3932x over previous
import jax
import jax.numpy as jnp
from jax.experimental import pallas as pl
from jax.experimental.pallas import tpu as pltpu

_N = 2048
_B = 8
_K = 10
_R = 256
_SWEEPS = 4
_JACOBI_ORDER = ((0, 2), (1, 2), (0, 1))


def _rotate(A, V, p, q):
    app, aqq, apq = A[p][p], A[q][q], A[p][q]
    tau = (aqq - app) / (2.0 * apq)
    t = jnp.sign(tau) / (jnp.abs(tau) + jnp.sqrt(1.0 + tau * tau))
    t = jnp.where(tau == 0.0, 1.0, t)
    c = 1.0 / jnp.sqrt(1.0 + t * t)
    s = t * c
    zero = apq == 0.0
    c = jnp.where(zero, 1.0, c)
    s = jnp.where(zero, 0.0, s)
    for j in range(3):
        ap, aq = A[p][j], A[q][j]
        A[p][j] = c * ap - s * aq
        A[q][j] = s * ap + c * aq
    for i in range(3):
        ap, aq = A[i][p], A[i][q]
        A[i][p] = c * ap - s * aq
        A[i][q] = s * ap + c * aq
    for i in range(3):
        vp, vq = V[i][p], V[i][q]
        V[i][p] = c * vp - s * vq
        V[i][q] = s * vp + c * vq


def _normal_from_sums(st):
    inv = 1.0 / st[9:10, :]
    mx = st[0:1, :] * inv
    my = st[1:2, :] * inv
    mz = st[2:3, :] * inv
    a00 = st[3:4, :] * inv - mx * mx
    a11 = st[4:5, :] * inv - my * my
    a22 = st[5:6, :] * inv - mz * mz
    a01 = st[6:7, :] * inv - mx * my
    a02 = st[7:8, :] * inv - mx * mz
    a12 = st[8:9, :] * inv - my * mz
    A = [[a00, a01, a02], [a01, a11, a12], [a02, a12, a22]]
    one = jnp.ones_like(a00)
    nil = jnp.zeros_like(a00)
    V = [[one, nil, nil], [nil, one, nil], [nil, nil, one]]
    for _ in range(_SWEEPS):
        for (p, q) in _JACOBI_ORDER:
            _rotate(A, V, p, q)
    d0, d1, d2 = A[0][0], A[1][1], A[2][2]
    b1 = d1 < d0
    best = jnp.where(b1, d1, d0)
    n = [jnp.where(b1, V[i][1], V[i][0]) for i in range(3)]
    b2 = d2 < best
    n = [jnp.where(b2, V[i][2], n[i]) for i in range(3)]
    norm = jnp.sqrt(n[0] * n[0] + n[1] * n[1] + n[2] * n[2]) + 1e-12
    return n[0] / norm, n[1] / norm, n[2] / norm


def _tile_normals(ref, t):
    P = ref[0]
    x, y, z = P[0:1, :], P[1:2, :], P[2:3, :]
    sq = x * x + y * y + z * z
    Prow = ref[0, :, pl.ds(t * _R, _R)]
    xr, yr, zr = Prow[0:1, :], Prow[1:2, :], Prow[2:3, :]
    sqr = xr * xr + yr * yr + zr * zr
    ones_r = jnp.ones_like(sqr)
    ones_n = jnp.ones_like(sq)
    lhs = jnp.concatenate([xr, yr, zr, ones_r, sqr], axis=0)
    rhs = jnp.concatenate([-2 * x, -2 * y, -2 * z, sq, ones_n], axis=0)
    d2 = jax.lax.dot_general(lhs, rhs, (((0,), (0,)), ((), ())),
                             preferred_element_type=jnp.float32)
    cur = d2
    thr = None
    for _ in range(_K):
        thr = jnp.min(cur, axis=1, keepdims=True)
        cur = jnp.where(cur <= thr, jnp.inf, cur)
    M = jnp.where(d2 <= thr, 1.0, 0.0)
    F = jnp.concatenate(
        [x, y, z, x * x, y * y, z * z, x * y, x * z, y * z, ones_n], axis=0)
    st = jax.lax.dot_general(F, M, (((1,), (1,)), ((), ())),
                             preferred_element_type=jnp.float32)
    return _normal_from_sums(st)


def _body(pred_ref, gt_ref, out_ref):
    b = pl.program_id(0)
    t = pl.program_id(1)
    p0, p1, p2 = _tile_normals(pred_ref, t)
    g0, g1, g2 = _tile_normals(gt_ref, t)
    e0, e1, e2 = p0 - g0, p1 - g1, p2 - g2
    ssq = jnp.sum(e0 * e0 + e1 * e1 + e2 * e2, keepdims=True)

    @pl.when((b == 0) & (t == 0))
    def _():
        out_ref[:, :] = jnp.zeros_like(ssq)

    out_ref[:, :] += ssq / float(_B * 3 * _N)


def kernel(pred, gt):
    out = pl.pallas_call(
        _body,
        grid=(_B, _N // _R),
        in_specs=[
            pl.BlockSpec((1, 3, _N), lambda b, t: (b, 0, 0)),
            pl.BlockSpec((1, 3, _N), lambda b, t: (b, 0, 0)),
        ],
        out_specs=pl.BlockSpec((1, 1), lambda b, t: (0, 0)),
        out_shape=jax.ShapeDtypeStruct((1, 1), jnp.float32),
        compiler_params=pltpu.CompilerParams(
            dimension_semantics=("arbitrary", "arbitrary")),
    )(pred, gt)
    return out[0, 0]
